# baseline (speedup 1.0000x reference)
import functools

import jax
import jax.numpy as jnp
import numpy as np
from jax import lax
from jax.experimental import pallas as pl
from jax.experimental.pallas import tpu as pltpu
from jax.experimental.pallas import tpu_sc as plsc

NN = 10000
EE = 320000
NG = 256
BN_EPS = 1e-5

NS = 16
EPT = EE // NS
C = 80
NCHUNK = EPT // C
ZR = 40
NZCHUNK = NN // ZR


@functools.lru_cache(maxsize=None)
def _edge_call(dh: int, split: bool):
    ept = EPT if split else EPT // 2
    nchunk = ept // C
    mesh = plsc.VectorSubcoreMesh(core_axis_name="c", subcore_axis_name="s")

    @functools.partial(
        pl.kernel,
        mesh=mesh,
        out_type=jax.ShapeDtypeStruct((2, NN, dh), jnp.float32),
        scratch_types=[
            pltpu.VMEM((3, C), jnp.int32),
            pltpu.VMEM((3, C), jnp.int32),
            pltpu.VMEM((2, C, dh), jnp.float32),
            pltpu.VMEM((2, C, dh), jnp.float32),
            pltpu.VMEM((ZR, dh), jnp.float32),
            pltpu.VMEM_SHARED((NN, dh), jnp.float32),
            pltpu.SemaphoreType.DMA,
            pltpu.SemaphoreType.DMA,
            pltpu.SemaphoreType.DMA,
            pltpu.SemaphoreType.DMA,
        ],
    )
    def edge_k(gsrc_hbm, dst3_hbm, e_hbm, xr_hbm, out_hbm,
               gidx_v, didx_v, ebuf, xbuf, zbuf, aggr_sh,
               sem_g, sem_e, sem_ig, sem_id):
        cid = lax.axis_index("c")
        sid = lax.axis_index("s")

        def zrow(r, carry):
            for ch in range(dh // 16):
                zbuf[r, pl.ds(ch * 16, 16)] = jnp.zeros((16,), jnp.float32)
            return carry

        lax.fori_loop(0, ZR, zrow, 0)
        for rep in range((NZCHUNK + NS - 1) // NS):
            ck = sid + NS * rep

            @pl.when(ck < NZCHUNK)
            def _():
                pltpu.sync_copy(zbuf, aggr_sh.at[pl.ds(ck * ZR, ZR), :])
        plsc.subcore_barrier()

        def e_slice(j):
            if split:
                return e_hbm.at[cid, pl.ds(sid * ept + j * C, C), :]
            return e_hbm.at[pl.ds(cid * (EE // 2) + sid * ept + j * C, C), :]

        def g_slice(j):
            if split:
                return gsrc_hbm.at[pl.ds(cid * EE + sid * ept + j * C, C)]
            return gsrc_hbm.at[pl.ds(cid * (EE // 2) + sid * ept + j * C, C)]

        def d_slice(j):
            if split:
                return dst3_hbm.at[sid, j]
            return dst3_hbm.at[cid, sid, j]

        pltpu.sync_copy(g_slice(0), gidx_v.at[0])
        pltpu.sync_copy(d_slice(0), didx_v.at[0])

        def body(j, carry):
            b = j % 2
            i3 = j % 3
            jc = jnp.minimum(j, nchunk - 1)

            @pl.when((j > 0) & (j < nchunk))
            def _():
                pltpu.make_async_copy(g_slice(jc), gidx_v.at[i3], sem_ig).wait()
                pltpu.make_async_copy(d_slice(jc), didx_v.at[i3], sem_id).wait()

            @pl.when(j < nchunk)
            def _():
                pltpu.async_copy(e_slice(jc), ebuf.at[b], sem_e)
                pltpu.async_copy(xr_hbm.at[gidx_v.at[i3]], xbuf.at[b], sem_g)

            @pl.when(j + 1 < nchunk)
            def _():
                jn = jnp.minimum(j + 1, nchunk - 1)
                pltpu.async_copy(g_slice(jn), gidx_v.at[(j + 1) % 3], sem_ig)
                pltpu.async_copy(d_slice(jn), didx_v.at[(j + 1) % 3], sem_id)

            @pl.when(j > 0)
            def _():
                bp = 1 - b
                p3 = (j + 2) % 3
                jp = jnp.maximum(j - 1, 0)
                pltpu.make_async_copy(e_slice(jp), ebuf.at[bp], sem_e).wait()
                pltpu.make_async_copy(xr_hbm.at[gidx_v.at[p3]], xbuf.at[bp],
                                      sem_g).wait()

                def rowf(r, rc):
                    for ch in range(dh // 16):
                        sl = pl.ds(ch * 16, 16)
                        ebuf[bp, r, sl] = jnp.maximum(
                            xbuf[bp, r, sl] + ebuf[bp, r, sl], 0.0)
                    return rc

                lax.fori_loop(0, C, rowf, 0)
                pltpu.sync_copy(ebuf.at[bp], aggr_sh.at[didx_v.at[p3]],
                                add=True)
            return carry

        lax.fori_loop(0, nchunk + 1, body, 0)
        plsc.subcore_barrier()

        for rep in range((NZCHUNK + NS - 1) // NS):
            ck = sid + NS * rep

            @pl.when(ck < NZCHUNK)
            def _():
                pltpu.sync_copy(aggr_sh.at[pl.ds(ck * ZR, ZR), :],
                                out_hbm.at[cid, pl.ds(ck * ZR, ZR), :])

    return edge_k


def _prep0_body(attr_ref, We_ref, be_ref, e_ref):
    e = lax.dot_general(attr_ref[...], We_ref[...], (((1,), (1,)), ((), ())),
                        preferred_element_type=jnp.float32)
    e_ref[...] = e + be_ref[...]


def _prep0(edge_attr, We0, be0):
    BE = 2000
    return pl.pallas_call(
        _prep0_body,
        grid=(EE // BE,),
        in_specs=[
            pl.BlockSpec((BE, 4), lambda i: (i, 0)),
            pl.BlockSpec((128, 4), lambda i: (0, 0)),
            pl.BlockSpec((1, 128), lambda i: (0, 0)),
        ],
        out_specs=pl.BlockSpec((BE, 128), lambda i: (i, 0)),
        out_shape=jax.ShapeDtypeStruct((EE, 128), jnp.float32),
    )(edge_attr, We0, be0.reshape(1, 128))


def _prep12_body(attr_ref, We1_ref, be1_ref, We2_ref, be2_ref, e1_ref, e2_ref):
    attr = attr_ref[...]
    for We_ref, be_ref, out_ref in ((We1_ref, be1_ref, e1_ref),
                                    (We2_ref, be2_ref, e2_ref)):
        e = lax.dot_general(attr, We_ref[...], (((1,), (1,)), ((), ())),
                            preferred_element_type=jnp.float32)
        out_ref[...] = (e + be_ref[0])[None]


def _prep12(edge_attr, We1, be1, We2, be2):
    BE = 2000
    return pl.pallas_call(
        _prep12_body,
        grid=(2, EE // BE),
        in_specs=[
            pl.BlockSpec((BE, 4), lambda h, i: (i, 0)),
            pl.BlockSpec((128, 4), lambda h, i: (h, 0)),
            pl.BlockSpec((1, 1, 128), lambda h, i: (h, 0, 0)),
            pl.BlockSpec((128, 4), lambda h, i: (h, 0)),
            pl.BlockSpec((1, 1, 128), lambda h, i: (h, 0, 0)),
        ],
        out_specs=[
            pl.BlockSpec((1, BE, 128), lambda h, i: (h, i, 0)),
            pl.BlockSpec((1, BE, 128), lambda h, i: (h, i, 0)),
        ],
        out_shape=[
            jax.ShapeDtypeStruct((2, EE, 128), jnp.float32),
            jax.ShapeDtypeStruct((2, EE, 128), jnp.float32),
        ],
    )(edge_attr, We1, be1.reshape(2, 1, 128), We2, be2.reshape(2, 1, 128))


def _mlp_body(split, x_ref, a_ref, W1_ref, b1_ref, W2_ref, b2_ref,
              g_ref, bt_ref, o_ref):
    x = x_ref[...]
    a = a_ref[...]
    W1 = W1_ref[...]
    if split:
        dh = x.shape[1] // 2
        z1 = (lax.dot_general(x[:, :dh] + a[0], W1[:, :dh],
                              (((1,), (1,)), ((), ())),
                              preferred_element_type=jnp.float32)
              + lax.dot_general(x[:, dh:] + a[1], W1[:, dh:],
                                (((1,), (1,)), ((), ())),
                                preferred_element_type=jnp.float32))
    else:
        z1 = lax.dot_general(x + a[0] + a[1], W1, (((1,), (1,)), ((), ())),
                             preferred_element_type=jnp.float32)
    z1 = jnp.maximum(z1 + b1_ref[...], 0.0)
    z2 = lax.dot_general(z1, W2_ref[...], (((1,), (1,)), ((), ())),
                         preferred_element_type=jnp.float32) + b2_ref[...]
    scale = g_ref[...] * np.float32(1.0 / np.sqrt(1.0 + BN_EPS))
    o_ref[...] = jnp.maximum(z2 * scale + bt_ref[...], 0.0)


def _mlp(x, aggr, W1, b1, W2, b2, gamma, beta, split):
    d = x.shape[1]
    BNODE = 400
    grid = (NN // BNODE,)
    full = lambda shape: pl.BlockSpec(shape, lambda i: (0, 0))
    return pl.pallas_call(
        functools.partial(_mlp_body, split),
        grid=grid,
        in_specs=[
            pl.BlockSpec((BNODE, d), lambda i: (i, 0)),
            pl.BlockSpec((2, BNODE, 128), lambda i: (0, i, 0)),
            full((256, d)), full((1, 256)),
            full((256, 256)), full((1, 256)),
            full((1, 256)), full((1, 256)),
        ],
        out_specs=pl.BlockSpec((BNODE, 256), lambda i: (i, 0)),
        out_shape=jax.ShapeDtypeStruct((NN, 256), jnp.float32),
    )(x, aggr, W1, b1.reshape(1, -1), W2, b2.reshape(1, -1),
      gamma.reshape(1, -1), beta.reshape(1, -1))


def _pool_body(h_ref, b_ref, Wp_ref, bp_ref, o_ref, g_acc):
    i = pl.program_id(0)
    ids = b_ref[0, 0, :]
    onehot = (ids[:, None] == lax.broadcasted_iota(jnp.int32, (200, NG), 1)
              ).astype(jnp.float32)
    contrib = lax.dot_general(onehot, h_ref[...], (((0,), (0,)), ((), ())),
                              preferred_element_type=jnp.float32)

    @pl.when(i == 0)
    def _():
        g_acc[...] = contrib

    @pl.when(i > 0)
    def _():
        g_acc[...] = g_acc[...] + contrib

    @pl.when(i == pl.num_programs(0) - 1)
    def _():
        out = lax.dot_general(g_acc[...], Wp_ref[...], (((1,), (1,)), ((), ())),
                              preferred_element_type=jnp.float32) + bp_ref[...]
        o_ref[...] = jnp.maximum(out, 0.0)


def _pool(h, batch3, Wp, bp):
    BP = 200
    grid = (NN // BP,)
    return pl.pallas_call(
        _pool_body,
        grid=grid,
        in_specs=[
            pl.BlockSpec((BP, 256), lambda i: (i, 0)),
            pl.BlockSpec((1, 1, BP), lambda i: (i, 0, 0)),
            pl.BlockSpec((128, 256), lambda i: (0, 0)),
            pl.BlockSpec((1, 128), lambda i: (0, 0)),
        ],
        out_specs=pl.BlockSpec((NG, 128), lambda i: (0, 0)),
        out_shape=jax.ShapeDtypeStruct((NG, 128), jnp.float32),
        scratch_shapes=[pltpu.VMEM((NG, 256), jnp.float32)],
    )(h, batch3, Wp, bp.reshape(1, -1))


def kernel(x, edge_index, edge_attr, batch,
           We0, be0, W1_0, b1_0, W2_0, b2_0, gamma0, beta0,
           We1, be1, W1_1, b1_1, W2_1, b2_1, gamma1, beta1,
           We2, be2, W1_2, b1_2, W2_2, b2_2, gamma2, beta2, Wp, bp):
    src = edge_index[0]
    dst = edge_index[1]
    idxs = jnp.concatenate([src * 2, src * 2 + 1])

    e0 = _prep0(edge_attr, We0, be0)
    e1, e2 = _prep12(edge_attr, We1, be1, We2, be2)

    dst_l0 = dst.reshape(2, NS, EPT // (2 * C), C)
    dst_sp = dst.reshape(NS, NCHUNK, C)

    a0 = _edge_call(128, False)(src, dst_l0, e0, x)
    h1 = _mlp(x, a0, W1_0, b1_0, W2_0, b2_0, gamma0, beta0, split=False)

    a1 = _edge_call(128, True)(idxs, dst_sp, e1, h1.reshape(2 * NN, 128))
    h2 = _mlp(h1, a1, W1_1, b1_1, W2_1, b2_1, gamma1, beta1, split=True)

    a2 = _edge_call(128, True)(idxs, dst_sp, e2, h2.reshape(2 * NN, 128))
    h3 = _mlp(h2, a2, W1_2, b1_2, W2_2, b2_2, gamma2, beta2, split=True)

    return _pool(h3, batch.reshape(NN // 200, 1, 200), Wp, bp)


# speedup vs baseline: 1.0021x; 1.0021x over previous
import functools

import jax
import jax.numpy as jnp
import numpy as np
from jax import lax
from jax.experimental import pallas as pl
from jax.experimental.pallas import tpu as pltpu
from jax.experimental.pallas import tpu_sc as plsc

NN = 10000
EE = 320000
NG = 256
BN_EPS = 1e-5

NS = 16
EPT = EE // NS
C = 80
NCHUNK = EPT // C
ZR = 40
NZCHUNK = NN // ZR


@functools.lru_cache(maxsize=None)
def _edge_call(dh: int, split: bool):
    ept = EPT if split else EPT // 2
    nchunk = ept // C
    mesh = plsc.VectorSubcoreMesh(core_axis_name="c", subcore_axis_name="s")

    @functools.partial(
        pl.kernel,
        mesh=mesh,
        out_type=jax.ShapeDtypeStruct((2, NN, dh), jnp.float32),
        scratch_types=[
            pltpu.VMEM((3, C), jnp.int32),
            pltpu.VMEM((3, C), jnp.int32),
            pltpu.VMEM((2, C, dh), jnp.float32),
            pltpu.VMEM((2, C, dh), jnp.float32),
            pltpu.VMEM((ZR, dh), jnp.float32),
            pltpu.VMEM_SHARED((NN, dh), jnp.float32),
            pltpu.SemaphoreType.DMA,
            pltpu.SemaphoreType.DMA,
            pltpu.SemaphoreType.DMA,
            pltpu.SemaphoreType.DMA,
            pltpu.SemaphoreType.DMA,
        ],
    )
    def edge_k(gsrc_hbm, dst3_hbm, e_hbm, xr_hbm, out_hbm,
               gidx_v, didx_v, ebuf, xbuf, zbuf, aggr_sh,
               sem_g, sem_e, sem_ig, sem_id, sem_s):
        cid = lax.axis_index("c")
        sid = lax.axis_index("s")

        def zrow(r, carry):
            for ch in range(dh // 16):
                zbuf[r, pl.ds(ch * 16, 16)] = jnp.zeros((16,), jnp.float32)
            return carry

        lax.fori_loop(0, ZR, zrow, 0)
        for rep in range((NZCHUNK + NS - 1) // NS):
            ck = sid + NS * rep

            @pl.when(ck < NZCHUNK)
            def _():
                pltpu.sync_copy(zbuf, aggr_sh.at[pl.ds(ck * ZR, ZR), :])
        plsc.subcore_barrier()

        def e_slice(j):
            if split:
                return e_hbm.at[cid, pl.ds(sid * ept + j * C, C), :]
            return e_hbm.at[pl.ds(cid * (EE // 2) + sid * ept + j * C, C), :]

        def g_slice(j):
            if split:
                return gsrc_hbm.at[pl.ds(cid * EE + sid * ept + j * C, C)]
            return gsrc_hbm.at[pl.ds(cid * (EE // 2) + sid * ept + j * C, C)]

        def d_slice(j):
            if split:
                return dst3_hbm.at[sid, j]
            return dst3_hbm.at[cid, sid, j]

        pltpu.sync_copy(g_slice(0), gidx_v.at[0])
        pltpu.sync_copy(d_slice(0), didx_v.at[0])

        def scat(slot, islot):
            return pltpu.make_async_copy(
                ebuf.at[slot], aggr_sh.at[didx_v.at[islot]], sem_s)

        def body(j, carry):
            b = j % 2
            i3 = j % 3
            jc = jnp.minimum(j, nchunk - 1)

            @pl.when((j > 1) & (j < nchunk))
            def _():
                scat(b, (j + 1) % 3).wait()

            @pl.when((j > 0) & (j < nchunk))
            def _():
                pltpu.make_async_copy(g_slice(jc), gidx_v.at[i3], sem_ig).wait()
                pltpu.make_async_copy(d_slice(jc), didx_v.at[i3], sem_id).wait()

            @pl.when(j < nchunk)
            def _():
                pltpu.async_copy(e_slice(jc), ebuf.at[b], sem_e)
                pltpu.async_copy(xr_hbm.at[gidx_v.at[i3]], xbuf.at[b], sem_g)

            @pl.when(j + 1 < nchunk)
            def _():
                jn = jnp.minimum(j + 1, nchunk - 1)
                pltpu.async_copy(g_slice(jn), gidx_v.at[(j + 1) % 3], sem_ig)
                pltpu.async_copy(d_slice(jn), didx_v.at[(j + 1) % 3], sem_id)

            @pl.when(j > 0)
            def _():
                bp = 1 - b
                p3 = (j + 2) % 3
                jp = jnp.maximum(j - 1, 0)
                pltpu.make_async_copy(e_slice(jp), ebuf.at[bp], sem_e).wait()
                pltpu.make_async_copy(xr_hbm.at[gidx_v.at[p3]], xbuf.at[bp],
                                      sem_g).wait()

                def rowf(r, rc):
                    for ch in range(dh // 16):
                        sl = pl.ds(ch * 16, 16)
                        ebuf[bp, r, sl] = jnp.maximum(
                            xbuf[bp, r, sl] + ebuf[bp, r, sl], 0.0)
                    return rc

                lax.fori_loop(0, C, rowf, 0)
                scat(bp, p3).start()
            return carry

        lax.fori_loop(0, nchunk + 1, body, 0)
        scat(0, 0).wait()
        scat(1, 1).wait()
        plsc.subcore_barrier()

        for rep in range((NZCHUNK + NS - 1) // NS):
            ck = sid + NS * rep

            @pl.when(ck < NZCHUNK)
            def _():
                pltpu.sync_copy(aggr_sh.at[pl.ds(ck * ZR, ZR), :],
                                out_hbm.at[cid, pl.ds(ck * ZR, ZR), :])

    return edge_k


def _prep0_body(attr_ref, We_ref, be_ref, e_ref):
    e = lax.dot_general(attr_ref[...], We_ref[...], (((1,), (1,)), ((), ())),
                        preferred_element_type=jnp.float32)
    e_ref[...] = e + be_ref[...]


def _prep0(edge_attr, We0, be0):
    BE = 2000
    return pl.pallas_call(
        _prep0_body,
        grid=(EE // BE,),
        in_specs=[
            pl.BlockSpec((BE, 4), lambda i: (i, 0)),
            pl.BlockSpec((128, 4), lambda i: (0, 0)),
            pl.BlockSpec((1, 128), lambda i: (0, 0)),
        ],
        out_specs=pl.BlockSpec((BE, 128), lambda i: (i, 0)),
        out_shape=jax.ShapeDtypeStruct((EE, 128), jnp.float32),
    )(edge_attr, We0, be0.reshape(1, 128))


def _prep12_body(attr_ref, We1_ref, be1_ref, We2_ref, be2_ref, e1_ref, e2_ref):
    attr = attr_ref[...]
    for We_ref, be_ref, out_ref in ((We1_ref, be1_ref, e1_ref),
                                    (We2_ref, be2_ref, e2_ref)):
        e = lax.dot_general(attr, We_ref[...], (((1,), (1,)), ((), ())),
                            preferred_element_type=jnp.float32)
        out_ref[...] = (e + be_ref[0])[None]


def _prep12(edge_attr, We1, be1, We2, be2):
    BE = 2000
    return pl.pallas_call(
        _prep12_body,
        grid=(2, EE // BE),
        in_specs=[
            pl.BlockSpec((BE, 4), lambda h, i: (i, 0)),
            pl.BlockSpec((128, 4), lambda h, i: (h, 0)),
            pl.BlockSpec((1, 1, 128), lambda h, i: (h, 0, 0)),
            pl.BlockSpec((128, 4), lambda h, i: (h, 0)),
            pl.BlockSpec((1, 1, 128), lambda h, i: (h, 0, 0)),
        ],
        out_specs=[
            pl.BlockSpec((1, BE, 128), lambda h, i: (h, i, 0)),
            pl.BlockSpec((1, BE, 128), lambda h, i: (h, i, 0)),
        ],
        out_shape=[
            jax.ShapeDtypeStruct((2, EE, 128), jnp.float32),
            jax.ShapeDtypeStruct((2, EE, 128), jnp.float32),
        ],
    )(edge_attr, We1, be1.reshape(2, 1, 128), We2, be2.reshape(2, 1, 128))


def _mlp_body(split, x_ref, a_ref, W1_ref, b1_ref, W2_ref, b2_ref,
              g_ref, bt_ref, o_ref):
    x = x_ref[...]
    a = a_ref[...]
    W1 = W1_ref[...]
    if split:
        dh = x.shape[1] // 2
        z1 = (lax.dot_general(x[:, :dh] + a[0], W1[:, :dh],
                              (((1,), (1,)), ((), ())),
                              preferred_element_type=jnp.float32)
              + lax.dot_general(x[:, dh:] + a[1], W1[:, dh:],
                                (((1,), (1,)), ((), ())),
                                preferred_element_type=jnp.float32))
    else:
        z1 = lax.dot_general(x + a[0] + a[1], W1, (((1,), (1,)), ((), ())),
                             preferred_element_type=jnp.float32)
    z1 = jnp.maximum(z1 + b1_ref[...], 0.0)
    z2 = lax.dot_general(z1, W2_ref[...], (((1,), (1,)), ((), ())),
                         preferred_element_type=jnp.float32) + b2_ref[...]
    scale = g_ref[...] * np.float32(1.0 / np.sqrt(1.0 + BN_EPS))
    o_ref[...] = jnp.maximum(z2 * scale + bt_ref[...], 0.0)


def _mlp(x, aggr, W1, b1, W2, b2, gamma, beta, split):
    d = x.shape[1]
    BNODE = 400
    grid = (NN // BNODE,)
    full = lambda shape: pl.BlockSpec(shape, lambda i: (0, 0))
    return pl.pallas_call(
        functools.partial(_mlp_body, split),
        grid=grid,
        in_specs=[
            pl.BlockSpec((BNODE, d), lambda i: (i, 0)),
            pl.BlockSpec((2, BNODE, 128), lambda i: (0, i, 0)),
            full((256, d)), full((1, 256)),
            full((256, 256)), full((1, 256)),
            full((1, 256)), full((1, 256)),
        ],
        out_specs=pl.BlockSpec((BNODE, 256), lambda i: (i, 0)),
        out_shape=jax.ShapeDtypeStruct((NN, 256), jnp.float32),
    )(x, aggr, W1, b1.reshape(1, -1), W2, b2.reshape(1, -1),
      gamma.reshape(1, -1), beta.reshape(1, -1))


def _pool_body(h_ref, b_ref, Wp_ref, bp_ref, o_ref, g_acc):
    i = pl.program_id(0)
    ids = b_ref[0, 0, :]
    onehot = (ids[:, None] == lax.broadcasted_iota(jnp.int32, (200, NG), 1)
              ).astype(jnp.float32)
    contrib = lax.dot_general(onehot, h_ref[...], (((0,), (0,)), ((), ())),
                              preferred_element_type=jnp.float32)

    @pl.when(i == 0)
    def _():
        g_acc[...] = contrib

    @pl.when(i > 0)
    def _():
        g_acc[...] = g_acc[...] + contrib

    @pl.when(i == pl.num_programs(0) - 1)
    def _():
        out = lax.dot_general(g_acc[...], Wp_ref[...], (((1,), (1,)), ((), ())),
                              preferred_element_type=jnp.float32) + bp_ref[...]
        o_ref[...] = jnp.maximum(out, 0.0)


def _pool(h, batch3, Wp, bp):
    BP = 200
    grid = (NN // BP,)
    return pl.pallas_call(
        _pool_body,
        grid=grid,
        in_specs=[
            pl.BlockSpec((BP, 256), lambda i: (i, 0)),
            pl.BlockSpec((1, 1, BP), lambda i: (i, 0, 0)),
            pl.BlockSpec((128, 256), lambda i: (0, 0)),
            pl.BlockSpec((1, 128), lambda i: (0, 0)),
        ],
        out_specs=pl.BlockSpec((NG, 128), lambda i: (0, 0)),
        out_shape=jax.ShapeDtypeStruct((NG, 128), jnp.float32),
        scratch_shapes=[pltpu.VMEM((NG, 256), jnp.float32)],
    )(h, batch3, Wp, bp.reshape(1, -1))


def kernel(x, edge_index, edge_attr, batch,
           We0, be0, W1_0, b1_0, W2_0, b2_0, gamma0, beta0,
           We1, be1, W1_1, b1_1, W2_1, b2_1, gamma1, beta1,
           We2, be2, W1_2, b1_2, W2_2, b2_2, gamma2, beta2, Wp, bp):
    src = edge_index[0]
    dst = edge_index[1]
    idxs = jnp.concatenate([src * 2, src * 2 + 1])

    e0 = _prep0(edge_attr, We0, be0)
    e1, e2 = _prep12(edge_attr, We1, be1, We2, be2)

    dst_l0 = dst.reshape(2, NS, EPT // (2 * C), C)
    dst_sp = dst.reshape(NS, NCHUNK, C)

    a0 = _edge_call(128, False)(src, dst_l0, e0, x)
    h1 = _mlp(x, a0, W1_0, b1_0, W2_0, b2_0, gamma0, beta0, split=False)

    a1 = _edge_call(128, True)(idxs, dst_sp, e1, h1.reshape(2 * NN, 128))
    h2 = _mlp(h1, a1, W1_1, b1_1, W2_1, b2_1, gamma1, beta1, split=True)

    a2 = _edge_call(128, True)(idxs, dst_sp, e2, h2.reshape(2 * NN, 128))
    h3 = _mlp(h2, a2, W1_2, b1_2, W2_2, b2_2, gamma2, beta2, split=True)

    return _pool(h3, batch.reshape(NN // 200, 1, 200), Wp, bp)


# speedup vs baseline: 1.4757x; 1.4726x over previous
import functools

import jax
import jax.numpy as jnp
import numpy as np
from jax import lax
from jax.experimental import pallas as pl
from jax.experimental.pallas import tpu as pltpu
from jax.experimental.pallas import tpu_sc as plsc

NN = 10000
EE = 320000
NG = 256
BN_EPS = 1e-5

NS = 16
EPT = EE // NS
C = 80
NCHUNK = EPT // C
NB = 25
ZR = 40
NZCHUNK = NN // ZR


@functools.lru_cache(maxsize=None)
def _edge_call(dh: int, split: bool):
    ept = EPT if split else EPT // 2
    nchunk = ept // C
    mesh = plsc.VectorSubcoreMesh(core_axis_name="c", subcore_axis_name="s")

    @functools.partial(
        pl.kernel,
        mesh=mesh,
        out_type=jax.ShapeDtypeStruct((2, NN, dh), jnp.float32),
        scratch_types=[
            pltpu.VMEM((NB * C,), jnp.int32),
            pltpu.VMEM((NB, C), jnp.int32),
            pltpu.VMEM((C, dh), jnp.float32),
            pltpu.VMEM((C, dh), jnp.float32),
            pltpu.VMEM((ZR, dh), jnp.float32),
            pltpu.VMEM_SHARED((NN, dh), jnp.float32),
            pltpu.SemaphoreType.DMA,
            pltpu.SemaphoreType.DMA,
        ],
    )
    def edge_k(gsrc_hbm, dst3_hbm, e_hbm, xr_hbm, out_hbm,
               gidx_v, didx_v, ebuf, xbuf, zbuf, aggr_sh,
               sem_g, sem_e):
        cid = lax.axis_index("c")
        sid = lax.axis_index("s")

        def zrow(r, carry):
            for ch in range(dh // 16):
                zbuf[r, pl.ds(ch * 16, 16)] = jnp.zeros((16,), jnp.float32)
            return carry

        lax.fori_loop(0, ZR, zrow, 0)
        for rep in range((NZCHUNK + NS - 1) // NS):
            ck = sid + NS * rep

            @pl.when(ck < NZCHUNK)
            def _():
                pltpu.sync_copy(zbuf, aggr_sh.at[pl.ds(ck * ZR, ZR), :])
        plsc.subcore_barrier()

        def body(bi, carry):
            if split:
                base = sid * ept + bi * NB * C
                g0 = cid * EE + base
                e_base = base
            else:
                base = cid * (EE // 2) + sid * ept + bi * NB * C
                g0 = base
                e_base = base
            pltpu.sync_copy(gsrc_hbm.at[pl.ds(g0, NB * C)], gidx_v)
            if split:
                pltpu.sync_copy(dst3_hbm.at[sid, bi], didx_v)
            else:
                pltpu.sync_copy(dst3_hbm.at[cid, sid, bi], didx_v)

            def chunk(k, kc):
                if split:
                    e_at = e_hbm.at[cid, pl.ds(e_base + k * C, C), :]
                else:
                    e_at = e_hbm.at[pl.ds(e_base + k * C, C), :]
                cp_e = pltpu.async_copy(e_at, ebuf, sem_e)
                cp_x = pltpu.async_copy(
                    xr_hbm.at[gidx_v.at[pl.ds(k * C, C)]], xbuf, sem_g)
                cp_e.wait()
                cp_x.wait()

                def rowf(r, rc):
                    for ch in range(dh // 16):
                        sl = pl.ds(ch * 16, 16)
                        ebuf[r, sl] = jnp.maximum(xbuf[r, sl] + ebuf[r, sl],
                                                  0.0)
                    return rc

                lax.fori_loop(0, C, rowf, 0)
                pltpu.sync_copy(ebuf, aggr_sh.at[didx_v.at[k]], add=True)
                return kc

            lax.fori_loop(0, NB, chunk, 0)
            return carry

        lax.fori_loop(0, ept // (NB * C), body, 0)
        plsc.subcore_barrier()

        for rep in range((NZCHUNK + NS - 1) // NS):
            ck = sid + NS * rep

            @pl.when(ck < NZCHUNK)
            def _():
                pltpu.sync_copy(aggr_sh.at[pl.ds(ck * ZR, ZR), :],
                                out_hbm.at[cid, pl.ds(ck * ZR, ZR), :])

    return edge_k


def _prep0_body(attr_ref, We_ref, be_ref, e_ref):
    e = lax.dot_general(attr_ref[...], We_ref[...], (((1,), (1,)), ((), ())),
                        preferred_element_type=jnp.float32)
    e_ref[...] = e + be_ref[...]


def _prep0(edge_attr, We0, be0):
    BE = 2000
    return pl.pallas_call(
        _prep0_body,
        grid=(EE // BE,),
        in_specs=[
            pl.BlockSpec((BE, 4), lambda i: (i, 0)),
            pl.BlockSpec((128, 4), lambda i: (0, 0)),
            pl.BlockSpec((1, 128), lambda i: (0, 0)),
        ],
        out_specs=pl.BlockSpec((BE, 128), lambda i: (i, 0)),
        out_shape=jax.ShapeDtypeStruct((EE, 128), jnp.float32),
    )(edge_attr, We0, be0.reshape(1, 128))


def _prep12_body(attr_ref, We1_ref, be1_ref, We2_ref, be2_ref, e1_ref, e2_ref):
    attr = attr_ref[...]
    for We_ref, be_ref, out_ref in ((We1_ref, be1_ref, e1_ref),
                                    (We2_ref, be2_ref, e2_ref)):
        e = lax.dot_general(attr, We_ref[...], (((1,), (1,)), ((), ())),
                            preferred_element_type=jnp.float32)
        out_ref[...] = (e + be_ref[0])[None]


def _prep12(edge_attr, We1, be1, We2, be2):
    BE = 2000
    return pl.pallas_call(
        _prep12_body,
        grid=(2, EE // BE),
        in_specs=[
            pl.BlockSpec((BE, 4), lambda h, i: (i, 0)),
            pl.BlockSpec((128, 4), lambda h, i: (h, 0)),
            pl.BlockSpec((1, 1, 128), lambda h, i: (h, 0, 0)),
            pl.BlockSpec((128, 4), lambda h, i: (h, 0)),
            pl.BlockSpec((1, 1, 128), lambda h, i: (h, 0, 0)),
        ],
        out_specs=[
            pl.BlockSpec((1, BE, 128), lambda h, i: (h, i, 0)),
            pl.BlockSpec((1, BE, 128), lambda h, i: (h, i, 0)),
        ],
        out_shape=[
            jax.ShapeDtypeStruct((2, EE, 128), jnp.float32),
            jax.ShapeDtypeStruct((2, EE, 128), jnp.float32),
        ],
    )(edge_attr, We1, be1.reshape(2, 1, 128), We2, be2.reshape(2, 1, 128))


def _mlp_body(split, x_ref, a_ref, W1_ref, b1_ref, W2_ref, b2_ref,
              g_ref, bt_ref, o_ref):
    x = x_ref[...]
    a = a_ref[...]
    W1 = W1_ref[...]
    if split:
        dh = x.shape[1] // 2
        z1 = (lax.dot_general(x[:, :dh] + a[0], W1[:, :dh],
                              (((1,), (1,)), ((), ())),
                              preferred_element_type=jnp.float32)
              + lax.dot_general(x[:, dh:] + a[1], W1[:, dh:],
                                (((1,), (1,)), ((), ())),
                                preferred_element_type=jnp.float32))
    else:
        z1 = lax.dot_general(x + a[0] + a[1], W1, (((1,), (1,)), ((), ())),
                             preferred_element_type=jnp.float32)
    z1 = jnp.maximum(z1 + b1_ref[...], 0.0)
    z2 = lax.dot_general(z1, W2_ref[...], (((1,), (1,)), ((), ())),
                         preferred_element_type=jnp.float32) + b2_ref[...]
    scale = g_ref[...] * np.float32(1.0 / np.sqrt(1.0 + BN_EPS))
    o_ref[...] = jnp.maximum(z2 * scale + bt_ref[...], 0.0)


def _mlp(x, aggr, W1, b1, W2, b2, gamma, beta, split):
    d = x.shape[1]
    BNODE = 400
    grid = (NN // BNODE,)
    full = lambda shape: pl.BlockSpec(shape, lambda i: (0, 0))
    return pl.pallas_call(
        functools.partial(_mlp_body, split),
        grid=grid,
        in_specs=[
            pl.BlockSpec((BNODE, d), lambda i: (i, 0)),
            pl.BlockSpec((2, BNODE, 128), lambda i: (0, i, 0)),
            full((256, d)), full((1, 256)),
            full((256, 256)), full((1, 256)),
            full((1, 256)), full((1, 256)),
        ],
        out_specs=pl.BlockSpec((BNODE, 256), lambda i: (i, 0)),
        out_shape=jax.ShapeDtypeStruct((NN, 256), jnp.float32),
    )(x, aggr, W1, b1.reshape(1, -1), W2, b2.reshape(1, -1),
      gamma.reshape(1, -1), beta.reshape(1, -1))


def _pool_body(h_ref, b_ref, Wp_ref, bp_ref, o_ref, g_acc):
    i = pl.program_id(0)
    ids = b_ref[0, 0, :]
    onehot = (ids[:, None] == lax.broadcasted_iota(jnp.int32, (200, NG), 1)
              ).astype(jnp.float32)
    contrib = lax.dot_general(onehot, h_ref[...], (((0,), (0,)), ((), ())),
                              preferred_element_type=jnp.float32)

    @pl.when(i == 0)
    def _():
        g_acc[...] = contrib

    @pl.when(i > 0)
    def _():
        g_acc[...] = g_acc[...] + contrib

    @pl.when(i == pl.num_programs(0) - 1)
    def _():
        out = lax.dot_general(g_acc[...], Wp_ref[...], (((1,), (1,)), ((), ())),
                              preferred_element_type=jnp.float32) + bp_ref[...]
        o_ref[...] = jnp.maximum(out, 0.0)


def _pool(h, batch3, Wp, bp):
    BP = 200
    grid = (NN // BP,)
    return pl.pallas_call(
        _pool_body,
        grid=grid,
        in_specs=[
            pl.BlockSpec((BP, 256), lambda i: (i, 0)),
            pl.BlockSpec((1, 1, BP), lambda i: (i, 0, 0)),
            pl.BlockSpec((128, 256), lambda i: (0, 0)),
            pl.BlockSpec((1, 128), lambda i: (0, 0)),
        ],
        out_specs=pl.BlockSpec((NG, 128), lambda i: (0, 0)),
        out_shape=jax.ShapeDtypeStruct((NG, 128), jnp.float32),
        scratch_shapes=[pltpu.VMEM((NG, 256), jnp.float32)],
    )(h, batch3, Wp, bp.reshape(1, -1))


def kernel(x, edge_index, edge_attr, batch,
           We0, be0, W1_0, b1_0, W2_0, b2_0, gamma0, beta0,
           We1, be1, W1_1, b1_1, W2_1, b2_1, gamma1, beta1,
           We2, be2, W1_2, b1_2, W2_2, b2_2, gamma2, beta2, Wp, bp):
    src = edge_index[0]
    dst = edge_index[1]
    idxs = jnp.concatenate([src * 2, src * 2 + 1])

    e0 = _prep0(edge_attr, We0, be0)
    e1, e2 = _prep12(edge_attr, We1, be1, We2, be2)

    dst_l0 = dst.reshape(2, NS, EPT // (2 * C * NB), NB, C)
    dst_sp = dst.reshape(NS, NCHUNK // NB, NB, C)

    a0 = _edge_call(128, False)(src, dst_l0, e0, x)
    h1 = _mlp(x, a0, W1_0, b1_0, W2_0, b2_0, gamma0, beta0, split=False)

    a1 = _edge_call(128, True)(idxs, dst_sp, e1, h1.reshape(2 * NN, 128))
    h2 = _mlp(h1, a1, W1_1, b1_1, W2_1, b2_1, gamma1, beta1, split=True)

    a2 = _edge_call(128, True)(idxs, dst_sp, e2, h2.reshape(2 * NN, 128))
    h3 = _mlp(h2, a2, W1_2, b1_2, W2_2, b2_2, gamma2, beta2, split=True)

    return _pool(h3, batch.reshape(NN // 200, 1, 200), Wp, bp)


# speedup vs baseline: 1.4821x; 1.0043x over previous
import functools

import jax
import jax.numpy as jnp
import numpy as np
from jax import lax
from jax.experimental import pallas as pl
from jax.experimental.pallas import tpu as pltpu
from jax.experimental.pallas import tpu_sc as plsc

NN = 10000
EE = 320000
NG = 256
BN_EPS = 1e-5

NS = 16
EPT = EE // NS
C = 80
NCHUNK = EPT // C
NB = 25
ZR = 40
NZCHUNK = NN // ZR
DR = 400
NDR = NN // DR


@functools.lru_cache(maxsize=None)
def _edge_call(dh: int, split: bool):
    ept = EPT if split else EPT // 2
    nchunk = ept // C
    mesh = plsc.VectorSubcoreMesh(core_axis_name="c", subcore_axis_name="s")

    @functools.partial(
        pl.kernel,
        mesh=mesh,
        out_type=jax.ShapeDtypeStruct((2, NN, dh), jnp.float32),
        scratch_types=[
            pltpu.VMEM((NB * C,), jnp.int32),
            pltpu.VMEM((NB, C), jnp.int32),
            pltpu.VMEM((C, dh), jnp.float32),
            pltpu.VMEM((C, dh), jnp.float32),
            pltpu.VMEM((ZR, dh), jnp.float32),
            pltpu.VMEM_SHARED((NN, dh), jnp.float32),
            pltpu.SemaphoreType.DMA,
            pltpu.SemaphoreType.DMA,
        ],
    )
    def edge_k(gsrc_hbm, dst3_hbm, e_hbm, xr_hbm, out_hbm,
               gidx_v, didx_v, ebuf, xbuf, zbuf, aggr_sh,
               sem_g, sem_e):
        cid = lax.axis_index("c")
        sid = lax.axis_index("s")

        def zrow(r, carry):
            for ch in range(dh // 16):
                zbuf[r, pl.ds(ch * 16, 16)] = jnp.zeros((16,), jnp.float32)
            return carry

        lax.fori_loop(0, ZR, zrow, 0)
        for rep in range((NZCHUNK + NS - 1) // NS):
            ck = sid + NS * rep

            @pl.when(ck < NZCHUNK)
            def _():
                pltpu.sync_copy(zbuf, aggr_sh.at[pl.ds(ck * ZR, ZR), :])
        plsc.subcore_barrier()

        def body(bi, carry):
            if split:
                base = sid * ept + bi * NB * C
                g0 = cid * EE + base
                e_base = base
            else:
                base = cid * (EE // 2) + sid * ept + bi * NB * C
                g0 = base
                e_base = base
            pltpu.sync_copy(gsrc_hbm.at[pl.ds(g0, NB * C)], gidx_v)
            if split:
                pltpu.sync_copy(dst3_hbm.at[sid, bi], didx_v)
            else:
                pltpu.sync_copy(dst3_hbm.at[cid, sid, bi], didx_v)

            def chunk(k, kc):
                if split:
                    e_at = e_hbm.at[cid, pl.ds(e_base + k * C, C), :]
                else:
                    e_at = e_hbm.at[pl.ds(e_base + k * C, C), :]
                cp_e = pltpu.async_copy(e_at, ebuf, sem_e)
                cp_x = pltpu.async_copy(
                    xr_hbm.at[gidx_v.at[pl.ds(k * C, C)]], xbuf, sem_g)
                cp_e.wait()
                cp_x.wait()

                def rowf(r4, rc):
                    for i in range(4):
                        r = r4 * 4 + i
                        for ch in range(dh // 16):
                            sl = pl.ds(ch * 16, 16)
                            ebuf[r, sl] = jnp.maximum(
                                xbuf[r, sl] + ebuf[r, sl], 0.0)
                    return rc

                lax.fori_loop(0, C // 4, rowf, 0)
                pltpu.sync_copy(ebuf, aggr_sh.at[didx_v.at[k]], add=True)
                return kc

            lax.fori_loop(0, NB, chunk, 0)
            return carry

        lax.fori_loop(0, ept // (NB * C), body, 0)
        plsc.subcore_barrier()

        for rep in range((NDR + NS - 1) // NS):
            ck = sid + NS * rep

            @pl.when(ck < NDR)
            def _():
                pltpu.sync_copy(aggr_sh.at[pl.ds(ck * DR, DR), :],
                                out_hbm.at[cid, pl.ds(ck * DR, DR), :])

    return edge_k


def _prep0_body(attr_ref, We_ref, be_ref, e_ref):
    e = lax.dot_general(attr_ref[...], We_ref[...], (((1,), (1,)), ((), ())),
                        preferred_element_type=jnp.float32)
    e_ref[...] = e + be_ref[...]


def _prep0(edge_attr, We0, be0):
    BE = 2000
    return pl.pallas_call(
        _prep0_body,
        grid=(EE // BE,),
        in_specs=[
            pl.BlockSpec((BE, 4), lambda i: (i, 0)),
            pl.BlockSpec((128, 4), lambda i: (0, 0)),
            pl.BlockSpec((1, 128), lambda i: (0, 0)),
        ],
        out_specs=pl.BlockSpec((BE, 128), lambda i: (i, 0)),
        out_shape=jax.ShapeDtypeStruct((EE, 128), jnp.float32),
    )(edge_attr, We0, be0.reshape(1, 128))


def _prep12_body(attr_ref, We1_ref, be1_ref, We2_ref, be2_ref, e1_ref, e2_ref):
    attr = attr_ref[...]
    for We_ref, be_ref, out_ref in ((We1_ref, be1_ref, e1_ref),
                                    (We2_ref, be2_ref, e2_ref)):
        e = lax.dot_general(attr, We_ref[...], (((1,), (1,)), ((), ())),
                            preferred_element_type=jnp.float32)
        out_ref[...] = (e + be_ref[0])[None]


def _prep12(edge_attr, We1, be1, We2, be2):
    BE = 2000
    return pl.pallas_call(
        _prep12_body,
        grid=(2, EE // BE),
        in_specs=[
            pl.BlockSpec((BE, 4), lambda h, i: (i, 0)),
            pl.BlockSpec((128, 4), lambda h, i: (h, 0)),
            pl.BlockSpec((1, 1, 128), lambda h, i: (h, 0, 0)),
            pl.BlockSpec((128, 4), lambda h, i: (h, 0)),
            pl.BlockSpec((1, 1, 128), lambda h, i: (h, 0, 0)),
        ],
        out_specs=[
            pl.BlockSpec((1, BE, 128), lambda h, i: (h, i, 0)),
            pl.BlockSpec((1, BE, 128), lambda h, i: (h, i, 0)),
        ],
        out_shape=[
            jax.ShapeDtypeStruct((2, EE, 128), jnp.float32),
            jax.ShapeDtypeStruct((2, EE, 128), jnp.float32),
        ],
    )(edge_attr, We1, be1.reshape(2, 1, 128), We2, be2.reshape(2, 1, 128))


def _mlp_body(split, x_ref, a_ref, W1_ref, b1_ref, W2_ref, b2_ref,
              g_ref, bt_ref, o_ref):
    x = x_ref[...]
    a = a_ref[...]
    W1 = W1_ref[...]
    if split:
        dh = x.shape[1] // 2
        z1 = (lax.dot_general(x[:, :dh] + a[0], W1[:, :dh],
                              (((1,), (1,)), ((), ())),
                              preferred_element_type=jnp.float32)
              + lax.dot_general(x[:, dh:] + a[1], W1[:, dh:],
                                (((1,), (1,)), ((), ())),
                                preferred_element_type=jnp.float32))
    else:
        z1 = lax.dot_general(x + a[0] + a[1], W1, (((1,), (1,)), ((), ())),
                             preferred_element_type=jnp.float32)
    z1 = jnp.maximum(z1 + b1_ref[...], 0.0)
    z2 = lax.dot_general(z1, W2_ref[...], (((1,), (1,)), ((), ())),
                         preferred_element_type=jnp.float32) + b2_ref[...]
    scale = g_ref[...] * np.float32(1.0 / np.sqrt(1.0 + BN_EPS))
    o_ref[...] = jnp.maximum(z2 * scale + bt_ref[...], 0.0)


def _mlp(x, aggr, W1, b1, W2, b2, gamma, beta, split):
    d = x.shape[1]
    BNODE = 400
    grid = (NN // BNODE,)
    full = lambda shape: pl.BlockSpec(shape, lambda i: (0, 0))
    return pl.pallas_call(
        functools.partial(_mlp_body, split),
        grid=grid,
        in_specs=[
            pl.BlockSpec((BNODE, d), lambda i: (i, 0)),
            pl.BlockSpec((2, BNODE, 128), lambda i: (0, i, 0)),
            full((256, d)), full((1, 256)),
            full((256, 256)), full((1, 256)),
            full((1, 256)), full((1, 256)),
        ],
        out_specs=pl.BlockSpec((BNODE, 256), lambda i: (i, 0)),
        out_shape=jax.ShapeDtypeStruct((NN, 256), jnp.float32),
    )(x, aggr, W1, b1.reshape(1, -1), W2, b2.reshape(1, -1),
      gamma.reshape(1, -1), beta.reshape(1, -1))


def _pool_body(h_ref, b_ref, Wp_ref, bp_ref, o_ref, g_acc):
    i = pl.program_id(0)
    ids = b_ref[0, 0, :]
    onehot = (ids[:, None] == lax.broadcasted_iota(jnp.int32, (200, NG), 1)
              ).astype(jnp.float32)
    contrib = lax.dot_general(onehot, h_ref[...], (((0,), (0,)), ((), ())),
                              preferred_element_type=jnp.float32)

    @pl.when(i == 0)
    def _():
        g_acc[...] = contrib

    @pl.when(i > 0)
    def _():
        g_acc[...] = g_acc[...] + contrib

    @pl.when(i == pl.num_programs(0) - 1)
    def _():
        out = lax.dot_general(g_acc[...], Wp_ref[...], (((1,), (1,)), ((), ())),
                              preferred_element_type=jnp.float32) + bp_ref[...]
        o_ref[...] = jnp.maximum(out, 0.0)


def _pool(h, batch3, Wp, bp):
    BP = 200
    grid = (NN // BP,)
    return pl.pallas_call(
        _pool_body,
        grid=grid,
        in_specs=[
            pl.BlockSpec((BP, 256), lambda i: (i, 0)),
            pl.BlockSpec((1, 1, BP), lambda i: (i, 0, 0)),
            pl.BlockSpec((128, 256), lambda i: (0, 0)),
            pl.BlockSpec((1, 128), lambda i: (0, 0)),
        ],
        out_specs=pl.BlockSpec((NG, 128), lambda i: (0, 0)),
        out_shape=jax.ShapeDtypeStruct((NG, 128), jnp.float32),
        scratch_shapes=[pltpu.VMEM((NG, 256), jnp.float32)],
    )(h, batch3, Wp, bp.reshape(1, -1))


def kernel(x, edge_index, edge_attr, batch,
           We0, be0, W1_0, b1_0, W2_0, b2_0, gamma0, beta0,
           We1, be1, W1_1, b1_1, W2_1, b2_1, gamma1, beta1,
           We2, be2, W1_2, b1_2, W2_2, b2_2, gamma2, beta2, Wp, bp):
    src = edge_index[0]
    dst = edge_index[1]
    idxs = jnp.concatenate([src * 2, src * 2 + 1])

    e0 = _prep0(edge_attr, We0, be0)
    e1, e2 = _prep12(edge_attr, We1, be1, We2, be2)

    dst_l0 = dst.reshape(2, NS, EPT // (2 * C * NB), NB, C)
    dst_sp = dst.reshape(NS, NCHUNK // NB, NB, C)

    a0 = _edge_call(128, False)(src, dst_l0, e0, x)
    h1 = _mlp(x, a0, W1_0, b1_0, W2_0, b2_0, gamma0, beta0, split=False)

    a1 = _edge_call(128, True)(idxs, dst_sp, e1, h1.reshape(2 * NN, 128))
    h2 = _mlp(h1, a1, W1_1, b1_1, W2_1, b2_1, gamma1, beta1, split=True)

    a2 = _edge_call(128, True)(idxs, dst_sp, e2, h2.reshape(2 * NN, 128))
    h3 = _mlp(h2, a2, W1_2, b1_2, W2_2, b2_2, gamma2, beta2, split=True)

    return _pool(h3, batch.reshape(NN // 200, 1, 200), Wp, bp)
